# baseline (speedup 1.0000x reference)
import functools

import jax
import jax.numpy as jnp
from jax import lax
from jax.experimental import pallas as pl
from jax.experimental.pallas import tpu as pltpu
from jax.experimental.pallas import tpu_sc as plsc

_NC, _NS, _L = 2, 16, 16
_NW = _NC * _NS

_CHUNK = 1024
_SUB = 128
_NPAD = 100096
_FBLK = 4352


def _sc_gather_half(table_h, fl3, h, C):
    P_h = 2 * _NPAD
    span = P_h // _NW
    n_chunks = -(-span // _CHUNK)
    nblk_total = fl3.shape[0] // 4
    roff = h * table_h.shape[0]

    mesh = plsc.VectorSubcoreMesh(core_axis_name="c", subcore_axis_name="s")

    @functools.partial(
        pl.kernel,
        out_type=jax.ShapeDtypeStruct((P_h, C), jnp.float32),
        mesh=mesh,
        compiler_params=pltpu.CompilerParams(
            needs_layout_passes=False, use_tc_tiling_on_sc=False
        ),
        scratch_types=[
            pltpu.VMEM((20, 128), jnp.int32),
            pltpu.VMEM((_CHUNK,), jnp.int32),
            pltpu.VMEM((_CHUNK, C), jnp.float32),
            pltpu.SemaphoreType.DMA,
        ],
    )
    def k(table_hbm, idx_hbm, out_hbm, win, fvm, rows, sem):
        w = lax.axis_index("s") * _NC + lax.axis_index("c")
        p0w = w * span
        lane = lax.iota(jnp.int32, _L)

        def chunk_body(c, carry):
            base = p0w + jnp.minimum(c * _CHUNK, span - _CHUNK)
            n0 = base // 2
            wblk = jnp.minimum(n0 // 128, nblk_total - 5)
            pltpu.sync_copy(idx_hbm.at[pl.ds(wblk * 4, 20)], win)

            def grp(g, carry2):
                j = g * _L + lane
                n = n0 + (j >> 1)
                rowi = ((n >> 7) - wblk) * 4 + (2 * h + (j & 1))
                fvm[pl.ds(g * _L, _L)] = jnp.maximum(
                    plsc.load_gather(win, [rowi, n & 127]) - roff, 0
                )
                return carry2

            lax.fori_loop(0, _CHUNK // _L, grp, 0)

            descs = [
                pltpu.async_copy(
                    table_hbm.at[fvm.at[pl.ds(kk * _SUB, _SUB)]],
                    rows.at[pl.ds(kk * _SUB, _SUB)],
                    sem,
                )
                for kk in range(_CHUNK // _SUB)
            ]
            for d in descs:
                d.wait()
            pltpu.sync_copy(rows, out_hbm.at[pl.ds(base, _CHUNK)])
            return carry

        lax.fori_loop(0, n_chunks, chunk_body, 0)

    return k(table_h, fl3)


def _fmt_body0(x_ref, o_ref):
    C = o_ref.shape[2]
    xT = x_ref[...].T
    o_ref[0, 0] = xT[0:C, :]
    o_ref[0, 1] = xT[C : 2 * C, :]


def _fmt_body1(x_ref, prev_ref, o_ref):
    del prev_ref
    _fmt_body0(x_ref, o_ref)


def _format0(pairs, C):
    nj = _NPAD // _FBLK
    return pl.pallas_call(
        _fmt_body0,
        grid=(nj,),
        in_specs=[pl.BlockSpec((_FBLK, 2 * C), lambda j: (j, 0))],
        out_specs=pl.BlockSpec((1, 2, C, _FBLK), lambda j: (0, 0, 0, j)),
        out_shape=jax.ShapeDtypeStruct((2, 2, C, _NPAD), jnp.float32),
    )(pairs)


def _format1(pairs, prev, C):
    nj = _NPAD // _FBLK
    return pl.pallas_call(
        _fmt_body1,
        grid=(nj,),
        in_specs=[
            pl.BlockSpec((_FBLK, 2 * C), lambda j: (j, 0)),
            pl.BlockSpec(memory_space=pl.ANY),
        ],
        out_specs=pl.BlockSpec((1, 2, C, _FBLK), lambda j: (1, 0, 0, j)),
        out_shape=jax.ShapeDtypeStruct((2, 2, C, _NPAD), jnp.float32),
        input_output_aliases={1: 0},
    )(pairs, prev)


def kernel(voxel_features, voxel_coords, num_points):
    B, C, D, H, W = voxel_features.shape
    N = voxel_coords.shape[1]
    V = D * H * W
    tableT = (
        voxel_features.reshape(B, C, V).transpose(0, 2, 1).reshape(B * V, C)
    )
    c32 = voxel_coords.astype(jnp.int32)
    fl = (
        c32[..., 0] * (H * W)
        + c32[..., 1] * W
        + c32[..., 2]
        + (jnp.arange(B, dtype=jnp.int32) * V)[:, None]
    )
    flp = jnp.pad(fl, ((0, 0), (0, _NPAD - N)))
    fl3 = (
        flp.reshape(B, _NPAD // 128, 128)
        .transpose(1, 0, 2)
        .reshape(B * (_NPAD // 128), 128)
    )
    t0 = tableT[0 : 2 * V]
    t1 = tableT[2 * V : 4 * V]
    out0 = _sc_gather_half(t0, fl3, 0, C)
    p0 = out0.reshape(_NPAD, 2 * C)
    out1 = _sc_gather_half(t1, fl3, 1, C)
    p1 = out1.reshape(_NPAD, 2 * C)
    acc = _format0(p0, C)
    outc = _format1(p1, acc, C)
    return outc[:, :, :, :N].reshape(B, C, N).transpose(0, 2, 1)
